# baseline (speedup 1.0000x reference)
import functools
import jax
import jax.numpy as jnp
from jax import lax
from jax.experimental import pallas as pl
from jax.experimental.pallas import tpu as pltpu
from jax.experimental.pallas import tpu_sc as plsc

TOKENS = 8192
D = 2048
E = 16
KH = 100
KP = 128
RBLK = 128
NP = TOKENS + E * RBLK
NB = NP // RBLK
NW = 32
TPW = TOKENS // NW
D2 = D // 2
CH = 16
NCH = TPW // CH


def _route_tc_body(idx_ref, idxt_ref, pos_ref, bexp_ref):
    idx_m = idx_ref[:, 0, :]
    parts = []
    for e in range(E):
        parts.append(jnp.sum((idx_m == e).astype(jnp.int32), axis=1,
                             keepdims=True))
    hist = jnp.concatenate(parts, axis=1)
    bparts = [jnp.zeros((1, E), jnp.int32)]
    for w in range(1, NW):
        bparts.append(jnp.sum(hist[:w], axis=0, keepdims=True))
    before = jnp.concatenate(bparts, axis=0)
    counts = jnp.sum(hist, axis=0, keepdims=True)
    padded = jnp.bitwise_and(counts + (RBLK - 1), jnp.int32(-RBLK))
    eparts = [jnp.zeros((1, 1), jnp.int32)]
    for e in range(1, E):
        eparts.append(jnp.sum(padded[:, :e], axis=1, keepdims=True))
    ebase = jnp.concatenate(eparts, axis=1)
    wbase = (ebase + before).astype(jnp.float32)

    rL = lax.broadcasted_iota(jnp.int32, (TPW, TPW), 0)
    cL = lax.broadcasted_iota(jnp.int32, (TPW, TPW), 1)
    ltri = (cL < rL).astype(jnp.bfloat16)
    ioE = lax.broadcasted_iota(jnp.int32, (TPW, E), 1)
    for w in range(NW):
        col = idxt_ref[w]
        ohb = (jnp.broadcast_to(col, (TPW, E)) == ioE)
        oh = ohb.astype(jnp.float32)
        prefix = jnp.dot(ltri, ohb.astype(jnp.bfloat16),
                         preferred_element_type=jnp.float32)
        rank = jnp.sum(prefix * oh, axis=1, keepdims=True)
        bsel = jnp.sum(oh * wbase[w:w + 1, :], axis=1, keepdims=True)
        pos_ref[w] = (rank + bsel + 0.5).astype(jnp.int32)

    blo = (lax.broadcasted_iota(jnp.int32, (NB, E), 0) * RBLK)
    cnt = jnp.sum((blo >= ebase).astype(jnp.int32), axis=1, keepdims=True)
    bexp_ref[...] = cnt - 1


def _scatter_x(pos_hbm, x_hbm, xs_hbm, pos_v, rows0, rows1, gsem0, gsem1, ssem0, ssem1):
    wid = lax.axis_index("s") * 2 + lax.axis_index("c")
    base = wid * TPW
    pltpu.sync_copy(pos_hbm.at[pl.ds(base, TPW)], pos_v)
    bufs = (rows0, rows1)
    g = [None] * NCH
    sc = [None] * NCH
    gsems = (gsem0, gsem1)
    ssems = (ssem0, ssem1)
    g[0] = pltpu.async_copy(x_hbm.at[pl.ds(base, CH)], rows0, gsem0)
    g[1] = pltpu.async_copy(x_hbm.at[pl.ds(base + CH, CH)], rows1, gsem1)
    for c in range(NCH):
        g[c].wait()
        pv = pos_v[pl.ds(c * CH, CH)]
        sc[c] = pltpu.async_copy(bufs[c % 2], xs_hbm.at[pv], ssems[c % 2])
        if c + 2 < NCH:
            sc[c].wait()
            g[c + 2] = pltpu.async_copy(
                x_hbm.at[pl.ds(base + (c + 2) * CH, CH)], bufs[c % 2],
                gsems[c % 2])
    sc[NCH - 2].wait()
    sc[NCH - 1].wait()


def _unpermute(pos_hbm, outs_hbm, out_hbm, pos_v, rows0, rows1, gsem0, gsem1, ssem0, ssem1):
    wid = lax.axis_index("s") * 2 + lax.axis_index("c")
    base = wid * TPW
    pltpu.sync_copy(pos_hbm.at[pl.ds(base, TPW)], pos_v)
    bufs = (rows0, rows1)
    g = [None] * NCH
    sc = [None] * NCH
    gsems = (gsem0, gsem1)
    ssems = (ssem0, ssem1)
    g[0] = pltpu.async_copy(outs_hbm.at[pos_v[pl.ds(0, CH)]], rows0, gsem0)
    g[1] = pltpu.async_copy(outs_hbm.at[pos_v[pl.ds(CH, CH)]], rows1, gsem1)
    for c in range(NCH):
        g[c].wait()
        sc[c] = pltpu.async_copy(
            bufs[c % 2], out_hbm.at[pl.ds(base + c * CH, CH)], ssems[c % 2])
        if c + 2 < NCH:
            sc[c].wait()
            g[c + 2] = pltpu.async_copy(
                outs_hbm.at[pos_v[pl.ds((c + 2) * CH, CH)]], bufs[c % 2],
                gsems[c % 2])
    sc[NCH - 2].wait()
    sc[NCH - 1].wait()


def _ffn_body(bexp_ref, xs_ref, wse_ref, wso_ref, bs_ref, w1_ref, b1_ref,
              w2_ref, b2_ref, o_ref):
    w = xs_ref[...]
    xe = lax.bitcast_convert_type(jnp.left_shift(w, 16), jnp.float32)
    xo = lax.bitcast_convert_type(
        jnp.bitwise_and(w, jnp.int32(-65536)), jnp.float32)
    ei = jnp.dot(xe.astype(jnp.bfloat16), wse_ref[...],
                 preferred_element_type=jnp.float32)
    ei = ei + jnp.dot(xo.astype(jnp.bfloat16), wso_ref[...],
                      preferred_element_type=jnp.float32)
    ei = jax.nn.relu(ei + bs_ref[...]).astype(jnp.bfloat16)
    h = jnp.dot(ei, w1_ref[0], preferred_element_type=jnp.float32)
    h = jax.nn.relu(h + b1_ref[0]).astype(jnp.bfloat16)
    o = jnp.dot(h, w2_ref[0], preferred_element_type=jnp.float32)
    o_ref[...] = o + b2_ref[0]


def kernel(x, idx, Ws, bs, W1, b1, W2, b2):
    idx = idx.astype(jnp.int32)
    idx3 = idx.reshape(NW, 1, TPW)
    idxt = idx.reshape(NW, TPW, 1)

    pos3, bexp2 = pl.pallas_call(
        _route_tc_body,
        out_shape=[
            jax.ShapeDtypeStruct((NW, TPW, 1), jnp.int32),
            jax.ShapeDtypeStruct((NB, 1), jnp.int32),
        ],
    )(idx3, idxt)
    pos = pos3.reshape(TOKENS)
    bexp = bexp2.reshape(NB)

    mesh = plsc.VectorSubcoreMesh(core_axis_name="c", subcore_axis_name="s")
    scatter_x = functools.partial(
        pl.kernel,
        mesh=mesh,
        out_type=jax.ShapeDtypeStruct((NP, D2), jnp.int32),
        scratch_types=[
            pltpu.VMEM((TPW,), jnp.int32),
            pltpu.VMEM((CH, D2), jnp.int32),
            pltpu.VMEM((CH, D2), jnp.int32),
            pltpu.SemaphoreType.DMA,
            pltpu.SemaphoreType.DMA,
            pltpu.SemaphoreType.DMA,
            pltpu.SemaphoreType.DMA,
        ],
    )(_scatter_x)
    xp = lax.bitcast_convert_type(
        x.astype(jnp.bfloat16).reshape(TOKENS, D2, 2), jnp.int32)
    xs = scatter_x(pos, xp)

    w1t = jnp.transpose(W1, (0, 2, 1)).astype(jnp.bfloat16)
    b1p = b1.reshape(E, 1, KH)
    w2t = jnp.transpose(W2, (0, 2, 1)).astype(jnp.bfloat16)
    b2r = b2.reshape(E, 1, D)

    grid_spec = pltpu.PrefetchScalarGridSpec(
        num_scalar_prefetch=1,
        grid=(NB,),
        in_specs=[
            pl.BlockSpec((RBLK, D2), lambda i, be: (i, 0)),
            pl.BlockSpec((D2, D), lambda i, be: (0, 0)),
            pl.BlockSpec((D2, D), lambda i, be: (0, 0)),
            pl.BlockSpec((1, D), lambda i, be: (0, 0)),
            pl.BlockSpec((1, D, KH), lambda i, be: (be[i], 0, 0)),
            pl.BlockSpec((1, 1, KH), lambda i, be: (be[i], 0, 0)),
            pl.BlockSpec((1, KH, D), lambda i, be: (be[i], 0, 0)),
            pl.BlockSpec((1, 1, D), lambda i, be: (be[i], 0, 0)),
        ],
        out_specs=pl.BlockSpec((RBLK, D), lambda i, be: (i, 0)),
    )
    outs = pl.pallas_call(
        _ffn_body,
        grid_spec=grid_spec,
        out_shape=jax.ShapeDtypeStruct((NP, D), jnp.float32),
        compiler_params=pltpu.CompilerParams(
            dimension_semantics=("arbitrary",)),
    )(bexp, xs, jnp.transpose(Ws[:, 0::2]).astype(jnp.bfloat16),
      jnp.transpose(Ws[:, 1::2]).astype(jnp.bfloat16),
      bs.reshape(1, D), w1t, b1p, w2t, b2r)

    unperm = functools.partial(
        pl.kernel,
        mesh=mesh,
        out_type=jax.ShapeDtypeStruct((TOKENS, D), jnp.float32),
        scratch_types=[
            pltpu.VMEM((TPW,), jnp.int32),
            pltpu.VMEM((CH, D), jnp.float32),
            pltpu.VMEM((CH, D), jnp.float32),
            pltpu.SemaphoreType.DMA,
            pltpu.SemaphoreType.DMA,
            pltpu.SemaphoreType.DMA,
            pltpu.SemaphoreType.DMA,
        ],
    )(_unpermute)
    return unperm(pos, outs)


# speedup vs baseline: 3.5135x; 3.5135x over previous
import functools
import jax
import jax.numpy as jnp
from jax import lax
from jax.experimental import pallas as pl
from jax.experimental.pallas import tpu as pltpu
from jax.experimental.pallas import tpu_sc as plsc

TOKENS = 8192
D = 2048
E = 16
KH = 100
KP = 128
RBLK = 128
NP = TOKENS + E * RBLK
NB = NP // RBLK
NW = 32
TPW = TOKENS // NW
CH = 16
NCH = TPW // CH


def _route_tc_body(idx_ref, idxt_ref, pos_ref, bexp_ref):
    idx_m = idx_ref[:, 0, :]
    parts = []
    for e in range(E):
        parts.append(jnp.sum((idx_m == e).astype(jnp.int32), axis=1,
                             keepdims=True))
    hist = jnp.concatenate(parts, axis=1)
    bparts = [jnp.zeros((1, E), jnp.int32)]
    for w in range(1, NW):
        bparts.append(jnp.sum(hist[:w], axis=0, keepdims=True))
    before = jnp.concatenate(bparts, axis=0)
    counts = jnp.sum(hist, axis=0, keepdims=True)
    padded = jnp.bitwise_and(counts + (RBLK - 1), jnp.int32(-RBLK))
    eparts = [jnp.zeros((1, 1), jnp.int32)]
    for e in range(1, E):
        eparts.append(jnp.sum(padded[:, :e], axis=1, keepdims=True))
    ebase = jnp.concatenate(eparts, axis=1)
    wbase = (ebase + before).astype(jnp.float32)

    rL = lax.broadcasted_iota(jnp.int32, (TPW, TPW), 0)
    cL = lax.broadcasted_iota(jnp.int32, (TPW, TPW), 1)
    ltri = (cL < rL).astype(jnp.bfloat16)
    ioE = lax.broadcasted_iota(jnp.int32, (TPW, E), 1)
    for w in range(NW):
        col = idxt_ref[w]
        ohb = (jnp.broadcast_to(col, (TPW, E)) == ioE)
        oh = ohb.astype(jnp.float32)
        prefix = jnp.dot(ltri, ohb.astype(jnp.bfloat16),
                         preferred_element_type=jnp.float32)
        rank = jnp.sum(prefix * oh, axis=1, keepdims=True)
        bsel = jnp.sum(oh * wbase[w:w + 1, :], axis=1, keepdims=True)
        pos_ref[w] = (rank + bsel + 0.5).astype(jnp.int32)

    blo = (lax.broadcasted_iota(jnp.int32, (NB, E), 0) * RBLK)
    cnt = jnp.sum((blo >= ebase).astype(jnp.int32), axis=1, keepdims=True)
    bexp_ref[...] = cnt - 1


def _scatter_x(pos_hbm, x_hbm, xs_hbm, pos_v, rows0, rows1, gsem0, gsem1, ssem0, ssem1):
    wid = lax.axis_index("s") * 2 + lax.axis_index("c")
    base = wid * TPW
    pltpu.sync_copy(pos_hbm.at[pl.ds(base, TPW)], pos_v)
    bufs = (rows0, rows1)
    g = [None] * NCH
    sc = [None] * NCH
    gsems = (gsem0, gsem1)
    ssems = (ssem0, ssem1)
    g[0] = pltpu.async_copy(x_hbm.at[pl.ds(base, CH)], rows0, gsem0)
    g[1] = pltpu.async_copy(x_hbm.at[pl.ds(base + CH, CH)], rows1, gsem1)
    for c in range(NCH):
        g[c].wait()
        pv = pos_v[pl.ds(c * CH, CH)]
        sc[c] = pltpu.async_copy(bufs[c % 2], xs_hbm.at[pv], ssems[c % 2])
        if c + 2 < NCH:
            sc[c].wait()
            g[c + 2] = pltpu.async_copy(
                x_hbm.at[pl.ds(base + (c + 2) * CH, CH)], bufs[c % 2],
                gsems[c % 2])
    sc[NCH - 2].wait()
    sc[NCH - 1].wait()


def _unpermute(pos_hbm, outs_hbm, out_hbm, pos_v, rows0, rows1, gsem0, gsem1, ssem0, ssem1):
    wid = lax.axis_index("s") * 2 + lax.axis_index("c")
    base = wid * TPW
    pltpu.sync_copy(pos_hbm.at[pl.ds(base, TPW)], pos_v)
    bufs = (rows0, rows1)
    g = [None] * NCH
    sc = [None] * NCH
    gsems = (gsem0, gsem1)
    ssems = (ssem0, ssem1)
    g[0] = pltpu.async_copy(outs_hbm.at[pos_v[pl.ds(0, CH)]], rows0, gsem0)
    g[1] = pltpu.async_copy(outs_hbm.at[pos_v[pl.ds(CH, CH)]], rows1, gsem1)
    for c in range(NCH):
        g[c].wait()
        sc[c] = pltpu.async_copy(
            bufs[c % 2], out_hbm.at[pl.ds(base + c * CH, CH)], ssems[c % 2])
        if c + 2 < NCH:
            sc[c].wait()
            g[c + 2] = pltpu.async_copy(
                outs_hbm.at[pos_v[pl.ds((c + 2) * CH, CH)]], bufs[c % 2],
                gsems[c % 2])
    sc[NCH - 2].wait()
    sc[NCH - 1].wait()


def _ffn_body(bexp_ref, xs_ref, wst_ref, bs_ref, w1_ref, b1_ref, w2_ref,
              b2_ref, o_ref):
    xb = xs_ref[...].astype(jnp.bfloat16)
    ei = jnp.dot(xb, wst_ref[...], preferred_element_type=jnp.float32)
    ei = jax.nn.relu(ei + bs_ref[...]).astype(jnp.bfloat16)
    h = jnp.dot(ei, w1_ref[0], preferred_element_type=jnp.float32)
    h = jax.nn.relu(h + b1_ref[0]).astype(jnp.bfloat16)
    o = jnp.dot(h, w2_ref[0], preferred_element_type=jnp.float32)
    o_ref[...] = o + b2_ref[0]


def kernel(x, idx, Ws, bs, W1, b1, W2, b2):
    idx = idx.astype(jnp.int32)
    idx3 = idx.reshape(NW, 1, TPW)
    idxt = idx.reshape(NW, TPW, 1)

    pos3, bexp2 = pl.pallas_call(
        _route_tc_body,
        out_shape=[
            jax.ShapeDtypeStruct((NW, TPW, 1), jnp.int32),
            jax.ShapeDtypeStruct((NB, 1), jnp.int32),
        ],
    )(idx3, idxt)
    pos = pos3.reshape(TOKENS)
    bexp = bexp2.reshape(NB)

    mesh = plsc.VectorSubcoreMesh(core_axis_name="c", subcore_axis_name="s")
    scatter_x = functools.partial(
        pl.kernel,
        mesh=mesh,
        out_type=jax.ShapeDtypeStruct((NP, D), jnp.float32),
        scratch_types=[
            pltpu.VMEM((TPW,), jnp.int32),
            pltpu.VMEM((CH, D), jnp.float32),
            pltpu.VMEM((CH, D), jnp.float32),
            pltpu.SemaphoreType.DMA,
            pltpu.SemaphoreType.DMA,
            pltpu.SemaphoreType.DMA,
            pltpu.SemaphoreType.DMA,
        ],
    )(_scatter_x)
    xs = scatter_x(pos, x)

    w1t = jnp.transpose(W1, (0, 2, 1)).astype(jnp.bfloat16)
    b1p = b1.reshape(E, 1, KH)
    w2t = jnp.transpose(W2, (0, 2, 1)).astype(jnp.bfloat16)
    b2r = b2.reshape(E, 1, D)

    grid_spec = pltpu.PrefetchScalarGridSpec(
        num_scalar_prefetch=1,
        grid=(NB,),
        in_specs=[
            pl.BlockSpec((RBLK, D), lambda i, be: (i, 0)),
            pl.BlockSpec((D, D), lambda i, be: (0, 0)),
            pl.BlockSpec((1, D), lambda i, be: (0, 0)),
            pl.BlockSpec((1, D, KH), lambda i, be: (be[i], 0, 0)),
            pl.BlockSpec((1, 1, KH), lambda i, be: (be[i], 0, 0)),
            pl.BlockSpec((1, KH, D), lambda i, be: (be[i], 0, 0)),
            pl.BlockSpec((1, 1, D), lambda i, be: (be[i], 0, 0)),
        ],
        out_specs=pl.BlockSpec((RBLK, D), lambda i, be: (i, 0)),
    )
    outs = pl.pallas_call(
        _ffn_body,
        grid_spec=grid_spec,
        out_shape=jax.ShapeDtypeStruct((NP, D), jnp.float32),
        compiler_params=pltpu.CompilerParams(
            dimension_semantics=("arbitrary",)),
    )(bexp, xs, Ws.T.astype(jnp.bfloat16), bs.reshape(1, D), w1t, b1p,
      w2t, b2r)

    unperm = functools.partial(
        pl.kernel,
        mesh=mesh,
        out_type=jax.ShapeDtypeStruct((TOKENS, D), jnp.float32),
        scratch_types=[
            pltpu.VMEM((TPW,), jnp.int32),
            pltpu.VMEM((CH, D), jnp.float32),
            pltpu.VMEM((CH, D), jnp.float32),
            pltpu.SemaphoreType.DMA,
            pltpu.SemaphoreType.DMA,
            pltpu.SemaphoreType.DMA,
            pltpu.SemaphoreType.DMA,
        ],
    )(_unpermute)
    return unperm(pos, outs)
